# Initial kernel scaffold; baseline (speedup 1.0000x reference)
import jax
import jax.numpy as jnp
from jax.experimental import pallas as pl


def kernel(x, x_t_slot, candidate_number, vecs_use, x_embedding_network, time_embeddings, W_seq1, b_seq1, W_seq2, b_seq2, W_out1, b_out1, W_out2, b_out2, W_in1, b_in1, W_in2, b_in2):
    raise NotImplementedError("write your pallas kernel here")


# speedup vs baseline: 9.8756x; 9.8756x over previous
import functools

import jax
import jax.numpy as jnp
from jax import lax
from jax.experimental import pallas as pl
from jax.experimental.pallas import tpu as pltpu
from jax.experimental.pallas import tpu_sc as plsc

DP = 32
TOPK = 10


def _sc_gather(vecs_pad, emb, x_flat, cand):
  info = plsc.get_sparse_core_info()
  nw = info.num_cores * info.num_subcores
  n = x_flat.shape[0]
  k = cand.shape[0]
  h = emb.shape[1]
  w = vecs_pad.shape[1]
  bn = n // nw
  bk = k // nw
  mesh = plsc.VectorSubcoreMesh(core_axis_name="c", subcore_axis_name="s")

  @functools.partial(
      pl.kernel,
      mesh=mesh,
      out_type=[
          jax.ShapeDtypeStruct((n, w), jnp.float32),
          jax.ShapeDtypeStruct((n, h), jnp.float32),
          jax.ShapeDtypeStruct((k, w), jnp.float32),
          jax.ShapeDtypeStruct((k, h), jnp.float32),
      ],
      scratch_types=[
          pltpu.VMEM((bn,), jnp.int32),
          pltpu.VMEM((bk,), jnp.int32),
          pltpu.VMEM((bn, w), jnp.float32),
          pltpu.VMEM((bn, h), jnp.float32),
          pltpu.VMEM((bk, w), jnp.float32),
          pltpu.VMEM((bk, h), jnp.float32),
          pltpu.SemaphoreType.DMA,
      ],
  )
  def gather(vecs_hbm, emb_hbm, xf_hbm, cand_hbm,
             xe_out, self_out, ne_out, ce_out,
             idx1_v, idx2_v, b_xe, b_self, b_ne, b_ce, sem):
    wid = lax.axis_index("s") * info.num_cores + lax.axis_index("c")
    base1 = wid * bn
    pltpu.sync_copy(xf_hbm.at[pl.ds(base1, bn)], idx1_v)
    pltpu.async_copy(vecs_hbm.at[idx1_v], b_xe, sem).wait()
    pltpu.sync_copy(b_xe, xe_out.at[pl.ds(base1, bn)])
    pltpu.async_copy(emb_hbm.at[idx1_v], b_self, sem).wait()
    pltpu.sync_copy(b_self, self_out.at[pl.ds(base1, bn)])
    base2 = wid * bk
    pltpu.sync_copy(cand_hbm.at[pl.ds(base2, bk)], idx2_v)
    pltpu.async_copy(vecs_hbm.at[idx2_v], b_ne, sem).wait()
    pltpu.sync_copy(b_ne, ne_out.at[pl.ds(base2, bk)])
    pltpu.async_copy(emb_hbm.at[idx2_v], b_ce, sem).wait()
    pltpu.sync_copy(b_ce, ce_out.at[pl.ds(base2, bk)])

  return gather(vecs_pad, emb, x_flat, cand)


_DEF = lax.Precision.DEFAULT
_HI = lax.Precision.HIGHEST


def _prep_body(seq, user, dv,
               xe3_ref, xt_ref, ne_ref, tpad_ref,
               w1_ref, b1_ref, w2_ref, b2_ref,
               wo1_ref, bo1_ref, wo2_ref, bo2_ref,
               wi1_ref, bi1_ref, wi2_ref, bi2_ref,
               q_ref, aa_ref, xi_ref, xin_ref, nt_ref, nen_ref, ntn_ref):
  n = seq * user
  xe3 = xe3_ref[...]
  pieces = []
  for j in range(5):
    sh = 4 - j
    if sh == 0:
      e = xe3
    else:
      e = jnp.concatenate([xe3[:sh], xe3[:seq - sh]], axis=0)
    pieces.append(e.reshape(n, DP)[:, :dv])
  hist = jnp.concatenate(pieces, axis=1)
  hid = jnp.maximum(
      jnp.dot(hist, w1_ref[...], preferred_element_type=jnp.float32,
              precision=_DEF) + b1_ref[...], 0.0)
  q = jnp.dot(hid, w2_ref[...], preferred_element_type=jnp.float32,
              precision=_DEF) + b2_ref[...]
  q_ref[...] = q.reshape(seq, user, DP)
  aa_ref[...] = jnp.sum(q * q, axis=1, keepdims=True).reshape(seq, user, 1)

  xt = xt_ref[...].reshape(n, 1)
  oh = jnp.where(xt == lax.broadcasted_iota(jnp.int32, (n, 24), 1), 1.0, 0.0)
  t_emb = jnp.dot(oh, tpad_ref[...], preferred_element_type=jnp.float32,
                  precision=_HI)
  xe2 = xe3.reshape(n, DP)
  comb = jnp.concatenate([xe2[:, :dv], t_emb[:, :dv]], axis=1)
  xh = jnp.maximum(
      jnp.dot(comb, wo1_ref[...], preferred_element_type=jnp.float32,
              precision=_DEF) + bo1_ref[...], 0.0)
  xi = jnp.dot(xh, wo2_ref[...], preferred_element_type=jnp.float32,
               precision=_DEF) + bo2_ref[...]
  xi_ref[...] = xi.reshape(seq, user, DP)
  xin_ref[...] = jnp.sum(xi * xi, axis=1, keepdims=True).reshape(seq, user, 1)

  ne = ne_ref[...]
  kk = ne.shape[0]
  t0b = jnp.broadcast_to(tpad_ref[0:1, :dv], (kk, dv))
  vo = jnp.concatenate([ne[:, :dv], t0b], axis=1)
  nh = jnp.maximum(
      jnp.dot(vo, wi1_ref[...], preferred_element_type=jnp.float32,
              precision=_DEF) + bi1_ref[...], 0.0)
  nt = jnp.dot(nh, wi2_ref[...], preferred_element_type=jnp.float32,
               precision=_DEF) + bi2_ref[...]
  nt_ref[...] = nt
  nen_ref[...] = jnp.sum(ne * ne, axis=1, keepdims=True)
  ntn_ref[...] = jnp.sum(nt * nt, axis=1, keepdims=True)


def _score_body(bs, kk, q_ref, aa_ref, xi_ref, xin_ref, se_ref,
                ne_ref, nt_ref, nen_ref, ntn_ref, ce_ref, o_ref):
  ab_e = lax.dot_general(q_ref[...], ne_ref[...], (((1,), (1,)), ((), ())),
                         preferred_element_type=jnp.float32, precision=_DEF)
  ab_t = lax.dot_general(xi_ref[...], nt_ref[...], (((1,), (1,)), ((), ())),
                         preferred_element_type=jnp.float32, precision=_DEF)
  d2e = aa_ref[...] - 2.0 * ab_e + nen_ref[...]
  d2t = xin_ref[...] - 2.0 * ab_t + ntn_ref[...]
  de = jnp.sqrt(jnp.maximum(d2e, 1e-12))
  dt = jnp.sqrt(jnp.maximum(d2t, 1e-12))
  score = jnp.exp(-0.015 * de - 0.005 * dt)

  iota = lax.broadcasted_iota(jnp.int32, (bs, kk), 1)
  cur = score
  vals = []
  idxs = []
  for _ in range(TOPK):
    m = jnp.max(cur, axis=1, keepdims=True)
    sel = cur == m
    ij = jnp.min(jnp.where(sel, iota, kk), axis=1, keepdims=True)
    vals.append(m)
    idxs.append(ij)
    cur = jnp.where(iota == ij, -1.0, cur)

  v11 = jnp.concatenate(vals + [jnp.ones((bs, 1), jnp.float32)], axis=1)
  mm = jnp.max(v11, axis=1, keepdims=True)
  ev = jnp.exp(v11 - mm)
  w = ev / jnp.sum(ev, axis=1, keepdims=True)

  wm = jnp.zeros((bs, kk), jnp.float32)
  for j in range(TOPK):
    wm = wm + jnp.where(iota == idxs[j], w[:, j:j + 1], 0.0)

  out = lax.dot_general(wm, ce_ref[...], (((1,), (0,)), ((), ())),
                        preferred_element_type=jnp.float32, precision=lax.Precision.HIGHEST)
  o_ref[...] = out + w[:, TOPK:TOPK + 1] * se_ref[...]


def _pad_mat(w, rows=DP, cols=DP):
  return jnp.zeros((rows, cols), jnp.float32).at[:w.shape[0], :w.shape[1]].set(w)


def _pad_vec(b):
  return jnp.pad(b, (0, DP - b.shape[0])).reshape(1, DP)


def kernel(x, x_t_slot, candidate_number, vecs_use, x_embedding_network,
           time_embeddings, W_seq1, b_seq1, W_seq2, b_seq2, W_out1, b_out1,
           W_out2, b_out2, W_in1, b_in1, W_in2, b_in2):
  seq, user = x.shape
  n = seq * user
  k = candidate_number.shape[0]
  dv = vecs_use.shape[1]
  h = x_embedding_network.shape[1]
  nseg = time_embeddings.shape[0]

  x_flat = x.reshape(-1)
  vecs_pad = jnp.pad(vecs_use, ((0, 0), (0, h - dv)))
  xe, self_emb, ne, cand_emb = _sc_gather(
      vecs_pad, x_embedding_network, x_flat, candidate_number)
  xe = xe[:, :DP]
  ne = ne[:, :DP]

  w1p = jnp.zeros((5 * dv, DP), jnp.float32).at[:, :dv].set(W_seq1)
  wo1p = jnp.zeros((2 * dv, DP), jnp.float32).at[:, :dv].set(W_out1)
  wi1p = jnp.zeros((2 * dv, DP), jnp.float32).at[:, :dv].set(W_in1)
  tpad = jnp.pad(time_embeddings, ((0, 0), (0, DP - dv)))

  ub = 32
  full = lambda shape: pl.BlockSpec(shape, lambda i: tuple(0 for _ in shape))
  prep = pl.pallas_call(
      functools.partial(_prep_body, seq, ub, dv),
      grid=(user // ub,),
      in_specs=[
          pl.BlockSpec((seq, ub, DP), lambda i: (0, i, 0)),
          pl.BlockSpec((seq, ub, 1), lambda i: (0, i, 0)),
          full((k, DP)), full((nseg, DP)),
          full((5 * dv, DP)), full((1, DP)), full((DP, DP)), full((1, DP)),
          full((2 * dv, DP)), full((1, DP)), full((DP, DP)), full((1, DP)),
          full((2 * dv, DP)), full((1, DP)), full((DP, DP)), full((1, DP)),
      ],
      out_specs=[
          pl.BlockSpec((seq, ub, DP), lambda i: (0, i, 0)),
          pl.BlockSpec((seq, ub, 1), lambda i: (0, i, 0)),
          pl.BlockSpec((seq, ub, DP), lambda i: (0, i, 0)),
          pl.BlockSpec((seq, ub, 1), lambda i: (0, i, 0)),
          full((k, DP)), full((k, 1)), full((k, 1)),
      ],
      out_shape=[
          jax.ShapeDtypeStruct((seq, user, DP), jnp.float32),
          jax.ShapeDtypeStruct((seq, user, 1), jnp.float32),
          jax.ShapeDtypeStruct((seq, user, DP), jnp.float32),
          jax.ShapeDtypeStruct((seq, user, 1), jnp.float32),
          jax.ShapeDtypeStruct((k, DP), jnp.float32),
          jax.ShapeDtypeStruct((k, 1), jnp.float32),
          jax.ShapeDtypeStruct((k, 1), jnp.float32),
      ],
  )
  q3, aa3, xi3, xin3, nt, nen, ntn = prep(
      xe.reshape(seq, user, DP), x_t_slot.reshape(seq, user, 1), ne, tpad,
      w1p, _pad_vec(b_seq1), _pad_mat(W_seq2), _pad_vec(b_seq2),
      wo1p, _pad_vec(b_out1), _pad_mat(W_out2), _pad_vec(b_out2),
      wi1p, _pad_vec(b_in1), _pad_mat(W_in2), _pad_vec(b_in2))
  q = q3.reshape(n, DP)
  aa = aa3.reshape(n, 1)
  xi = xi3.reshape(n, DP)
  xin = xin3.reshape(n, 1)
  nen_row = nen.reshape(1, k)
  ntn_row = ntn.reshape(1, k)

  bs = 256
  grid = (n // bs,)
  score = pl.pallas_call(
      functools.partial(_score_body, bs, k),
      grid=grid,
      in_specs=[
          pl.BlockSpec((bs, DP), lambda i: (i, 0)),
          pl.BlockSpec((bs, 1), lambda i: (i, 0)),
          pl.BlockSpec((bs, DP), lambda i: (i, 0)),
          pl.BlockSpec((bs, 1), lambda i: (i, 0)),
          pl.BlockSpec((bs, h), lambda i: (i, 0)),
          pl.BlockSpec((k, DP), lambda i: (0, 0)),
          pl.BlockSpec((k, DP), lambda i: (0, 0)),
          pl.BlockSpec((1, k), lambda i: (0, 0)),
          pl.BlockSpec((1, k), lambda i: (0, 0)),
          pl.BlockSpec((k, h), lambda i: (0, 0)),
      ],
      out_specs=pl.BlockSpec((bs, h), lambda i: (i, 0)),
      out_shape=jax.ShapeDtypeStruct((n, h), jnp.float32),
  )
  return score(q, aa, xi, xin, self_emb, ne, nt, nen_row, ntn_row, cand_emb)
